# Initial kernel scaffold; baseline (speedup 1.0000x reference)
import jax
import jax.numpy as jnp
from jax.experimental import pallas as pl


def kernel(output, mask, ind, target):
    raise NotImplementedError("write your pallas kernel here")


# speedup vs baseline: 2.2583x; 2.2583x over previous
import jax
import jax.numpy as jnp
from jax import lax
from jax.experimental import pallas as pl
from jax.experimental.pallas import tpu as pltpu
from jax.experimental.pallas import tpu_sc as plsc

_B, _C, _H, _W = 32, 2, 512, 512
_HW = _H * _W
_CHW = _C * _HW
_K = 500
_KPAD = 512
_NIDX = _C * _KPAD
_LANES = 16


def _tec_body(flat_ref, ind_ref, mask_ref, tgt_ref, out_ref,
              ind_v, mask_v, tgt_v, idx_v, vals_v, part_v, sem):
    c = lax.axis_index("c")
    s = lax.axis_index("s")
    b = s * 2 + c

    pltpu.sync_copy(ind_ref.at[b], ind_v)
    pltpu.sync_copy(mask_ref.at[b], mask_v)
    pltpu.sync_copy(tgt_ref.at[b], tgt_v)

    base = b * _CHW
    for i in range(_KPAD // _LANES):
        sl = pl.ds(i * _LANES, _LANES)
        iv = ind_v[sl] + base
        idx_v[sl] = iv
        idx_v[pl.ds(_KPAD + i * _LANES, _LANES)] = iv + _HW

    copies = []
    for j in range(_NIDX // 128):
        sl = pl.ds(j * 128, 128)
        copies.append(pltpu.async_copy(flat_ref.at[idx_v.at[sl]],
                                       vals_v.at[sl], sem))
    for cp in copies:
        cp.wait()

    acc = jnp.zeros((_LANES,), jnp.float32)
    macc = jnp.zeros((_LANES,), jnp.float32)
    for i in range(_KPAD // _LANES):
        sl0 = pl.ds(i * _LANES, _LANES)
        sl1 = pl.ds(_KPAD + i * _LANES, _LANES)
        m = mask_v[sl0].astype(jnp.float32)
        d0 = jnp.abs(vals_v[sl0] - tgt_v[sl0])
        d1 = jnp.abs(vals_v[sl1] - tgt_v[sl1])
        acc = acc + (d0 + d1) * m
        macc = macc + m

    part_v[pl.ds(0, _LANES)] = acc
    part_v[pl.ds(_LANES, _LANES)] = macc
    pltpu.sync_copy(part_v, out_ref.at[b])


@jax.jit
def kernel(output, mask, ind, target):
    flat = output.reshape(_B * _CHW)
    ind_p = jnp.zeros((_B, _KPAD), jnp.int32).at[:, :_K].set(ind)
    mask_p = jnp.zeros((_B, _KPAD), jnp.int32).at[:, :_K].set(
        mask.astype(jnp.int32))
    tgt_p = jnp.zeros((_B, _C, _KPAD), jnp.float32).at[:, :, :_K].set(
        jnp.transpose(target, (0, 2, 1))).reshape(_B, _C * _KPAD)

    mesh = plsc.VectorSubcoreMesh(core_axis_name="c", subcore_axis_name="s")
    f = pl.kernel(
        _tec_body,
        mesh=mesh,
        out_type=jax.ShapeDtypeStruct((_B, 2 * _LANES), jnp.float32),
        scratch_types=[
            pltpu.VMEM((_KPAD,), jnp.int32),
            pltpu.VMEM((_KPAD,), jnp.int32),
            pltpu.VMEM((_NIDX,), jnp.float32),
            pltpu.VMEM((_NIDX,), jnp.int32),
            pltpu.VMEM((_NIDX,), jnp.float32),
            pltpu.VMEM((2 * _LANES,), jnp.float32),
            pltpu.SemaphoreType.DMA,
        ],
    )
    parts = f(flat, ind_p, mask_p, tgt_p)
    loss = jnp.sum(parts[:, :_LANES]) / (
        _C * jnp.sum(parts[:, _LANES:]) + 1e-4)
    return loss


# speedup vs baseline: 5.0081x; 2.2176x over previous
import jax
import jax.numpy as jnp
from jax import lax
from jax.experimental import pallas as pl
from jax.experimental.pallas import tpu as pltpu
from jax.experimental.pallas import tpu_sc as plsc

_B, _C, _H, _W = 32, 2, 512, 512
_K = 500
_KPAD = 512
_NIDX = _C * _KPAD
_LANES = 16
_BLK = 8


def _tec_body(feat_ref, ind_ref, mask_ref, tgt_ref, out_ref, stage_ref,
              ind_v, mask_v, tgt_v, vals8_v, pos_v, vals_v, part_v,
              sem, sem2):
    c = lax.axis_index("c")
    s = lax.axis_index("s")
    b = s * 2 + c

    pltpu.sync_copy(ind_ref.at[b], ind_v)
    pltpu.sync_copy(mask_ref.at[b], mask_v)
    pltpu.sync_copy(tgt_ref.at[b], tgt_v)

    row0_base = b * _C * _H

    def issue(i, carry):
        iv = ind_v[pl.ds(pl.multiple_of(i * _LANES, _LANES), _LANES)]
        for j in range(_LANES):
            p = iv[j]
            k = i * _LANES + j
            h = lax.shift_right_logical(p, 9)
            w8 = pl.multiple_of(p & (_W - 1) & ~(_BLK - 1), _BLK)
            dst0 = pl.multiple_of(k * _BLK, _BLK)
            dst1 = pl.multiple_of((_KPAD + k) * _BLK, _BLK)
            r0 = row0_base + h
            pltpu.async_copy(feat_ref.at[r0, pl.ds(w8, _BLK)],
                             vals8_v.at[pl.ds(dst0, _BLK)], sem)
            pltpu.async_copy(feat_ref.at[r0 + _H, pl.ds(w8, _BLK)],
                             vals8_v.at[pl.ds(dst1, _BLK)], sem)
        return carry

    lax.fori_loop(0, _KPAD // _LANES, issue, 0)

    lane_ids = lax.iota(jnp.int32, _LANES)
    for i in range(_KPAD // _LANES):
        sl = pl.ds(i * _LANES, _LANES)
        l = ind_v[sl] & (_BLK - 1)
        k_vec = lane_ids + i * _LANES + b * _NIDX
        pos_v[sl] = k_vec * _BLK + l
        pos_v[pl.ds(_KPAD + i * _LANES, _LANES)] = (
            (k_vec + _KPAD) * _BLK + l)

    for j in range(_NIDX * _BLK // _W):
        pltpu.make_async_copy(feat_ref.at[0],
                              vals8_v.at[pl.ds(j * _W, _W)], sem).wait()

    pltpu.sync_copy(vals8_v,
                    stage_ref.at[pl.ds(b * _NIDX * _BLK, _NIDX * _BLK)])
    copies = []
    for j in range(_NIDX // 128):
        sl = pl.ds(j * 128, 128)
        copies.append(pltpu.async_copy(stage_ref.at[pos_v.at[sl]],
                                       vals_v.at[sl], sem2))
    for cp in copies:
        cp.wait()

    acc = jnp.zeros((_LANES,), jnp.float32)
    macc = jnp.zeros((_LANES,), jnp.float32)
    for i in range(_KPAD // _LANES):
        sl0 = pl.ds(i * _LANES, _LANES)
        sl1 = pl.ds(_KPAD + i * _LANES, _LANES)
        m = mask_v[sl0].astype(jnp.float32)
        d0 = jnp.abs(vals_v[sl0] - tgt_v[sl0])
        d1 = jnp.abs(vals_v[sl1] - tgt_v[sl1])
        acc = acc + (d0 + d1) * m
        macc = macc + m

    part_v[pl.ds(0, _LANES)] = acc
    part_v[pl.ds(_LANES, _LANES)] = macc
    pltpu.sync_copy(part_v, out_ref.at[b])


@jax.jit
def kernel(output, mask, ind, target):
    feat = output.reshape(_B * _C * _H, _W)
    ind_p = jnp.zeros((_B, _KPAD), jnp.int32).at[:, :_K].set(ind)
    mask_p = jnp.zeros((_B, _KPAD), jnp.int32).at[:, :_K].set(
        mask.astype(jnp.int32))
    tgt_p = jnp.zeros((_B, _C, _KPAD), jnp.float32).at[:, :, :_K].set(
        jnp.transpose(target, (0, 2, 1))).reshape(_B, _C * _KPAD)

    mesh = plsc.VectorSubcoreMesh(core_axis_name="c", subcore_axis_name="s")
    f = pl.kernel(
        _tec_body,
        mesh=mesh,
        out_type=(
            jax.ShapeDtypeStruct((_B, 2 * _LANES), jnp.float32),
            jax.ShapeDtypeStruct((_B * _NIDX * _BLK,), jnp.float32),
        ),
        scratch_types=[
            pltpu.VMEM((_KPAD,), jnp.int32),
            pltpu.VMEM((_KPAD,), jnp.int32),
            pltpu.VMEM((_NIDX,), jnp.float32),
            pltpu.VMEM((_NIDX * _BLK,), jnp.float32),
            pltpu.VMEM((_NIDX,), jnp.int32),
            pltpu.VMEM((_NIDX,), jnp.float32),
            pltpu.VMEM((2 * _LANES,), jnp.float32),
            pltpu.SemaphoreType.DMA,
            pltpu.SemaphoreType.DMA,
        ],
    )
    parts, _ = f(feat, ind_p, mask_p, tgt_p)
    loss = jnp.sum(parts[:, :_LANES]) / (
        _C * jnp.sum(parts[:, _LANES:]) + 1e-4)
    return loss
